# baseline (speedup 1.0000x reference)
import functools

import jax
import jax.numpy as jnp
from jax import lax
from jax.experimental import pallas as pl
from jax.experimental.pallas import tpu as pltpu
from jax.experimental.pallas import tpu_sc as plsc

_N = 10000
_E = 320000
_IN = 166
_H = 76
_CH = 510
_SLOPE = (1.0 / 8.0 + 1.0 / 3.0) / 2.0

_NP = 10240
_HP = 80
_NCORE = 2
_NSUB = 16
_NW = _NCORE * _NSUB
_EP = 327680
_CHK = 128
_RPT = _NP // _NSUB
_FCH = 126
_SCH = 160 - _FCH
_FAST_CORE = 0

_mesh = plsc.VectorSubcoreMesh(core_axis_name="c", subcore_axis_name="s",
                               num_cores=_NCORE, num_subcores=_NSUB)


def _mm(a, b):
    return lax.dot_general(a, b, (((1,), (0,)), ((), ())),
                           precision=lax.Precision.DEFAULT,
                           preferred_element_type=jnp.float32)


def _mmT(a, b):
    return lax.dot_general(a, b, (((0,), (0,)), ((), ())),
                           precision=lax.Precision.HIGHEST,
                           preferred_element_type=jnp.float32)


def _rrelu(x):
    return jnp.where(x >= 0, x, x * _SLOPE)


@functools.partial(
    pl.kernel,
    out_type=jax.ShapeDtypeStruct((2, 2, _NP, 16), jnp.float32),
    mesh=_mesh,
    compiler_params=pltpu.CompilerParams(use_tc_tiling_on_sc=False),
    scratch_types=[
        pltpu.VMEM_SHARED((_NP, 16), jnp.float32),
        pltpu.VMEM_SHARED((_NP, 16), jnp.float32),
        pltpu.VMEM((_FCH, 1, _CHK), jnp.int32),
        pltpu.VMEM((_FCH, 1, _CHK), jnp.int32),
        pltpu.VMEM((_CHK, 16), jnp.float32),
        pltpu.SemaphoreType.DMA,
        pltpu.SemaphoreType.DMA,
    ],
)
def _sc_degrees(src3_hbm, dst3_hbm, zeros16_hbm, out_hbm,
                sbuf, dbuf, sidxs, didxs, ones_v, sem0, sem1):
    c = lax.axis_index("c")
    s = lax.axis_index("s")
    wid = c * _NSUB + s
    nch = jnp.where(c == _FAST_CORE, _FCH, _SCH)

    def fill(i, _):
        ones_v[i] = jnp.ones((16,), jnp.float32)
        return 0
    lax.fori_loop(0, _CHK, fill, 0)

    pltpu.sync_copy(src3_hbm.at[wid], sidxs)
    pltpu.sync_copy(dst3_hbm.at[wid], didxs)
    pltpu.sync_copy(zeros16_hbm.at[pl.ds(s * _RPT, _RPT)],
                    sbuf.at[pl.ds(s * _RPT, _RPT)])
    pltpu.sync_copy(zeros16_hbm.at[pl.ds(s * _RPT, _RPT)],
                    dbuf.at[pl.ds(s * _RPT, _RPT)])
    plsc.subcore_barrier()

    def body(g, _):
        a = pltpu.async_copy(ones_v, sbuf.at[sidxs.at[g, 0]], sem0, add=True)
        b = pltpu.async_copy(ones_v, dbuf.at[didxs.at[g, 0]], sem1, add=True)
        a.wait()
        b.wait()
        return 0
    lax.fori_loop(0, nch, body, 0)
    plsc.subcore_barrier()

    pltpu.sync_copy(sbuf.at[pl.ds(s * _RPT, _RPT)],
                    out_hbm.at[c, 0, pl.ds(s * _RPT, _RPT)])
    pltpu.sync_copy(dbuf.at[pl.ds(s * _RPT, _RPT)],
                    out_hbm.at[c, 1, pl.ds(s * _RPT, _RPT)])


@functools.partial(
    pl.kernel,
    out_type=jax.ShapeDtypeStruct((2, _NP, _HP), jnp.float32),
    mesh=_mesh,
    compiler_params=pltpu.CompilerParams(use_tc_tiling_on_sc=False),
    scratch_types=[
        pltpu.VMEM_SHARED((_NP, _HP), jnp.float32),
        pltpu.VMEM((_FCH, 1, _CHK), jnp.int32),
        pltpu.VMEM((_FCH, 1, _CHK), jnp.int32),
        pltpu.VMEM((_CHK, _HP), jnp.float32),
        pltpu.VMEM((_CHK, _HP), jnp.float32),
        pltpu.SemaphoreType.DMA,
        pltpu.SemaphoreType.DMA,
    ],
)
def _sc_conv(h_hbm, src3_hbm, dst3_hbm, zeros_hbm, out_hbm,
             agg, sidxs, didxs, rows0, rows1, sem0, sem1):
    c = lax.axis_index("c")
    s = lax.axis_index("s")
    wid = c * _NSUB + s
    nch = jnp.where(c == _FAST_CORE, _FCH, _SCH)

    pltpu.sync_copy(src3_hbm.at[wid], sidxs)
    pltpu.sync_copy(dst3_hbm.at[wid], didxs)
    pltpu.async_copy(h_hbm.at[sidxs.at[0, 0]], rows0, sem0)
    pltpu.async_copy(h_hbm.at[sidxs.at[1, 0]], rows1, sem1)
    pltpu.sync_copy(zeros_hbm.at[pl.ds(s * _RPT, _RPT)],
                    agg.at[pl.ds(s * _RPT, _RPT)])
    plsc.subcore_barrier()

    def body(t, _):
        g0 = t * 2
        g1 = g0 + 1
        pltpu.make_async_copy(h_hbm.at[sidxs.at[g0, 0]], rows0, sem0).wait()
        pltpu.sync_copy(rows0, agg.at[didxs.at[g0, 0]], add=True)

        @pl.when(g1 + 1 < nch)
        def _():
            pltpu.async_copy(h_hbm.at[sidxs.at[g1 + 1, 0]], rows0, sem0)

        pltpu.make_async_copy(h_hbm.at[sidxs.at[g1, 0]], rows1, sem1).wait()
        pltpu.sync_copy(rows1, agg.at[didxs.at[g1, 0]], add=True)

        @pl.when(g1 + 2 < nch)
        def _():
            pltpu.async_copy(h_hbm.at[sidxs.at[g1 + 2, 0]], rows1, sem1)
        return 0
    lax.fori_loop(0, nch // 2, body, 0)
    plsc.subcore_barrier()

    pltpu.sync_copy(agg.at[pl.ds(s * _RPT, _RPT)],
                    out_hbm.at[c, pl.ds(s * _RPT, _RPT)])


def _topk_z(X, scorer):
    R = X.shape[1]
    nrm = jnp.maximum(jnp.sqrt(jnp.sum(scorer * scorer)), 1e-6)
    sw = _mm(X, jnp.broadcast_to(scorer, (R, 128)))
    eye = (lax.broadcasted_iota(jnp.int32, (128, 128), 0)
           == lax.broadcasted_iota(jnp.int32, (128, 128), 1)
           ).astype(jnp.float32)
    st = jnp.concatenate(
        [jnp.sum(sw[a * 128:(a + 1) * 128, :] * eye, axis=0, keepdims=True)
         for a in range(_NP // 128)], axis=0) / nrm
    ni = lax.broadcasted_iota(jnp.int32, (_NP, 1), 0)
    nid = (lax.broadcasted_iota(jnp.int32, (_NP // 128, 128), 0) * 128
           + lax.broadcasted_iota(jnp.int32, (_NP // 128, 128), 1))
    st = jnp.where(nid < _N, st, -jnp.inf)
    kl = lax.broadcasted_iota(jnp.int32, (1, 128), 1)

    def it(k, carry):
        stc, ia, va = carry
        m = jnp.max(stc)
        i = jnp.min(jnp.where(stc == m, nid, _NP))
        ia = jnp.where(kl == k, i, ia)
        va = jnp.where(kl == k, m, va)
        stc = jnp.where(nid == i, -jnp.inf, stc)
        return stc, ia, va

    _, ia, va = lax.fori_loop(
        0, _H, it,
        (st, jnp.zeros((1, 128), jnp.int32), jnp.zeros((1, 128), jnp.float32)))
    sel = ia[:, :_H]
    vals = va[:, :_H]
    Pt = jnp.where(ni == sel, jnp.tanh(vals), 0.0)
    return _mmT(X, Pt)


def _gru(z, w, uw, uu, ub, rw, ru, rb, hw, hu, hb):
    upd = jax.nn.sigmoid(_mm(uw, z) + _mm(uu, w) + ub)
    rst = jax.nn.sigmoid(_mm(rw, z) + _mm(ru, w) + rb)
    hc = jnp.tanh(_mm(hw, z) + _mm(hu, rst * w) + hb)
    return (1.0 - upd) * w + upd * hc


def _pad_w(w):
    return jnp.concatenate([w, jnp.zeros((w.shape[0], _HP - _H), jnp.float32)],
                           axis=1)


def _deg_body(po_ref, pi_ref, qo_ref, qi_ref, dout_ref, din_ref):
    dout_ref[...] = lax.rsqrt(jnp.maximum(po_ref[...] + qo_ref[...], 1.0))
    din_ref[...] = lax.rsqrt(jnp.maximum(pi_ref[...] + qi_ref[...], 1.0))


def _prep_body(x_ref, sc_ref, uw_ref, uu_ref, ub_ref, rw_ref, ru_ref, rb_ref,
               hw_ref, hu_ref, hb_ref, w0_ref, dout_ref,
               w01_ref, h0_ref):
    X = x_ref[...]
    z = _topk_z(X, sc_ref[...])
    w01 = _gru(z, w0_ref[...], uw_ref[...], uu_ref[...], ub_ref[...],
               rw_ref[...], ru_ref[...], rb_ref[...],
               hw_ref[...], hu_ref[...], hb_ref[...])
    w01_ref[...] = w01
    h0_ref[...] = _mm(X * dout_ref[...][:, 0:1], _pad_w(w01))


def _stepb_body(x_ref, dout_ref, sc_ref, uw_ref, uu_ref, ub_ref, rw_ref,
                ru_ref, rb_ref, hw_ref, hu_ref, hb_ref, w01_ref, h1_ref):
    X = x_ref[...]
    z = _topk_z(X, sc_ref[...])
    w02 = _gru(z, w01_ref[...], uw_ref[...], uu_ref[...], ub_ref[...],
               rw_ref[...], ru_ref[...], rb_ref[...],
               hw_ref[...], hu_ref[...], hb_ref[...])
    h1_ref[...] = _mm(X * dout_ref[...][:, 0:1], _pad_w(w02))


def _stepc_body(a0_ref, a1_ref, din_ref, sc_ref,
                uw_ref, uu_ref, ub_ref, rw_ref, ru_ref, rb_ref, hw_ref,
                hu_ref, hb_ref, w1_ref, w11_ref):
    A0 = _rrelu((a0_ref[...] + a1_ref[...]) * din_ref[...][:, 0:1])
    z0 = _topk_z(A0[:, :_H], sc_ref[...])
    w11_ref[...] = _gru(z0, w1_ref[...], uw_ref[...], uu_ref[...], ub_ref[...],
                        rw_ref[...], ru_ref[...], rb_ref[...],
                        hw_ref[...], hu_ref[...], hb_ref[...])


def _stepd_body(b0_ref, b1_ref, din_ref, dout_ref, sc_ref,
                uw_ref, uu_ref, ub_ref, rw_ref, ru_ref, rb_ref, hw_ref,
                hu_ref, hb_ref, w11_ref, h3_ref):
    A1 = _rrelu((b0_ref[...] + b1_ref[...]) * din_ref[...][:, 0:1])
    z1 = _topk_z(A1[:, :_H], sc_ref[...])
    w12 = _gru(z1, w11_ref[...], uw_ref[...], uu_ref[...], ub_ref[...],
               rw_ref[...], ru_ref[...], rb_ref[...],
               hw_ref[...], hu_ref[...], hb_ref[...])
    h3_ref[...] = _mm(A1[:, :_H] * dout_ref[...][:, 0:1], _pad_w(w12))


def _final_body(c0_ref, c1_ref, din_ref, w1_ref, b1_ref, w2_ref, b2_ref,
                out_ref):
    B = _rrelu((c0_ref[...] + c1_ref[...]) * din_ref[...][:, 0:1])[:, :_H]
    hh = jnp.maximum(_mm(B, w1_ref[...]) + b1_ref[...], 0.0)
    w2p = jnp.concatenate(
        [w2_ref[...], jnp.zeros((_CH, 126), jnp.float32)], axis=1)
    out_ref[...] = _mm(hh, w2p)[:, :2] + b2_ref[...]


_sds = jax.ShapeDtypeStruct
_tc_params = pltpu.CompilerParams(vmem_limit_bytes=128 * 1024 * 1024, fuse_transposed_lhs_in_matmul=True)

_tc_deg = pl.pallas_call(
    _deg_body,
    out_shape=(_sds((_NP // 8, 128), jnp.float32),
               _sds((_NP // 8, 128), jnp.float32)),
    compiler_params=_tc_params)

_tc_prep = pl.pallas_call(
    _prep_body,
    out_shape=(_sds((_IN, _H), jnp.float32), _sds((_NP, _HP), jnp.float32)),
    compiler_params=_tc_params)

_tc_stepb = pl.pallas_call(
    _stepb_body, out_shape=_sds((_NP, _HP), jnp.float32),
    compiler_params=_tc_params)

_tc_stepc = pl.pallas_call(
    _stepc_body, out_shape=_sds((_H, _H), jnp.float32),
    compiler_params=_tc_params)

_tc_stepd = pl.pallas_call(
    _stepd_body, out_shape=_sds((_NP, _HP), jnp.float32),
    compiler_params=_tc_params)

_tc_final = pl.pallas_call(
    _final_body, out_shape=_sds((_NP, 2), jnp.float32),
    compiler_params=_tc_params)


def kernel(feat, edge_index, scorer0, gru0_uw, gru0_uu, gru0_ub, gru0_rw,
           gru0_ru, gru0_rb, gru0_hw, gru0_hu, gru0_hb, W0, scorer1, gru1_uw,
           gru1_uu, gru1_ub, gru1_rw, gru1_ru, gru1_rb, gru1_hw, gru1_hu,
           gru1_hb, W1, mlp_w1, mlp_b1, mlp_w2, mlp_b2):
    def _edge_layout(flat):
        nfast = _NSUB * _FCH * _CHK
        fast = flat[:nfast].reshape(_NSUB, _FCH, 1, _CHK)
        slow = jnp.pad(flat[nfast:].reshape(_NSUB, _SCH, 1, _CHK),
                       ((0, 0), (0, _FCH - _SCH), (0, 0), (0, 0)))
        parts = [fast, slow] if _FAST_CORE == 0 else [slow, fast]
        return jnp.concatenate(parts, axis=0)

    pad_e = jnp.full((_EP - _E,), _NP - 1, jnp.int32)
    srcp = _edge_layout(jnp.concatenate([edge_index[0], pad_e]))
    dstp = _edge_layout(jnp.concatenate([edge_index[1], pad_e]))
    feat0p = jnp.pad(feat[0], ((0, _NP - _N), (0, 0)))
    feat1p = jnp.pad(feat[1], ((0, _NP - _N), (0, 0)))
    zeros80 = jnp.zeros((_NP, _HP), jnp.float32)
    zeros16 = jnp.zeros((_NP, 16), jnp.float32)

    degp = _sc_degrees(srcp, dstp, zeros16)
    dout_pk, din_pk = _tc_deg(
        degp[0, 0].reshape(_NP // 8, 128), degp[0, 1].reshape(_NP // 8, 128),
        degp[1, 0].reshape(_NP // 8, 128), degp[1, 1].reshape(_NP // 8, 128))
    dout16 = dout_pk.reshape(_NP, 16)
    din16 = din_pk.reshape(_NP, 16)
    w01, h0 = _tc_prep(
        feat0p, scorer0, gru0_uw, gru0_uu, gru0_ub, gru0_rw, gru0_ru,
        gru0_rb, gru0_hw, gru0_hu, gru0_hb, W0, dout16)
    aggA = _sc_conv(h0, srcp, dstp, zeros80)
    h1 = _tc_stepb(feat1p, dout16, scorer0, gru0_uw, gru0_uu, gru0_ub,
                   gru0_rw, gru0_ru, gru0_rb, gru0_hw, gru0_hu, gru0_hb, w01)
    aggB = _sc_conv(h1, srcp, dstp, zeros80)
    w11 = _tc_stepc(aggA[0], aggA[1], din16, scorer1, gru1_uw, gru1_uu,
                    gru1_ub, gru1_rw, gru1_ru, gru1_rb, gru1_hw, gru1_hu,
                    gru1_hb, W1)
    h3 = _tc_stepd(aggB[0], aggB[1], din16, dout16, scorer1, gru1_uw,
                   gru1_uu, gru1_ub, gru1_rw, gru1_ru, gru1_rb, gru1_hw,
                   gru1_hu, gru1_hb, w11)
    aggC = _sc_conv(h3, srcp, dstp, zeros80)
    out = _tc_final(aggC[0], aggC[1], din16, mlp_w1,
                    mlp_b1.reshape(1, _CH), mlp_w2, mlp_b2.reshape(1, 2))
    return out[:_N]


# speedup vs baseline: 1.8476x; 1.8476x over previous
import functools

import jax
import jax.numpy as jnp
from jax import lax
from jax.experimental import pallas as pl
from jax.experimental.pallas import tpu as pltpu
from jax.experimental.pallas import tpu_sc as plsc

_N = 10000
_E = 320000
_IN = 166
_H = 76
_CH = 510
_SLOPE = (1.0 / 8.0 + 1.0 / 3.0) / 2.0

_NP = 10240
_HP = 80
_NCORE = 2
_NSUB = 16
_NW = _NCORE * _NSUB
_EP = 327680
_CHK = 128
_RPT = _NP // _NSUB
_FCH = 80
_SCH = 160 - _FCH
_FAST_CORE = 0
_NCHUNK = 80
_PAD = _N
_SPN = 10016
_RPS = _SPN // _NSUB

_mesh = plsc.VectorSubcoreMesh(core_axis_name="c", subcore_axis_name="s",
                               num_cores=_NCORE, num_subcores=_NSUB)


def _mm(a, b):
    return lax.dot_general(a, b, (((1,), (0,)), ((), ())),
                           precision=lax.Precision.DEFAULT,
                           preferred_element_type=jnp.float32)


def _mmT(a, b):
    return lax.dot_general(a, b, (((0,), (0,)), ((), ())),
                           precision=lax.Precision.HIGHEST,
                           preferred_element_type=jnp.float32)


def _rrelu(x):
    return jnp.where(x >= 0, x, x * _SLOPE)


@functools.partial(
    pl.kernel,
    out_type=jax.ShapeDtypeStruct((2, 2, _NP, 16), jnp.float32),
    mesh=_mesh,
    compiler_params=pltpu.CompilerParams(use_tc_tiling_on_sc=False),
    scratch_types=[
        pltpu.VMEM_SHARED((_NP, 16), jnp.float32),
        pltpu.VMEM((_FCH, 1, _CHK), jnp.int32),
        pltpu.VMEM((_FCH, 1, _CHK), jnp.int32),
        pltpu.VMEM((_CHK, 16), jnp.float32),
        pltpu.SemaphoreType.DMA,
        pltpu.SemaphoreType.DMA,
    ],
)
def _sc_degrees(src3_hbm, dst3_hbm, zeros16_hbm, out_hbm,
                sbuf, sidxs, didxs, ones_v, sem0, sem1):
    c = lax.axis_index("c")
    s = lax.axis_index("s")
    wid = c * _NSUB + s
    nch = jnp.where(c == _FAST_CORE, _FCH, _SCH)

    def fill(i, _):
        ones_v[i] = jnp.ones((16,), jnp.float32)
        return 0
    lax.fori_loop(0, _CHK, fill, 0)

    pltpu.sync_copy(src3_hbm.at[wid], sidxs)
    pltpu.sync_copy(dst3_hbm.at[wid], didxs)

    for which, idxs in ((0, sidxs), (1, didxs)):
        pltpu.sync_copy(zeros16_hbm.at[pl.ds(s * _RPT, _RPT)],
                        sbuf.at[pl.ds(s * _RPT, _RPT)])
        plsc.subcore_barrier()

        def body(g, _):
            a = pltpu.async_copy(ones_v, sbuf.at[idxs.at[g, 0]], sem0,
                                 add=True)
            b = pltpu.async_copy(ones_v, sbuf.at[idxs.at[g + 1, 0]], sem1,
                                 add=True)
            a.wait()
            b.wait()
            return 0
        lax.fori_loop(0, nch // 2, lambda g, z, body=body: body(2 * g, z), 0)
        plsc.subcore_barrier()
        pltpu.sync_copy(sbuf.at[pl.ds(s * _RPT, _RPT)],
                        out_hbm.at[c, which, pl.ds(s * _RPT, _RPT)])
        plsc.subcore_barrier()


@functools.partial(
    pl.kernel,
    out_type=jax.ShapeDtypeStruct((2, _NP, _HP), jnp.float32),
    mesh=_mesh,
    compiler_params=pltpu.CompilerParams(use_tc_tiling_on_sc=False),
    scratch_types=[
        pltpu.VMEM_SHARED((_SPN, _HP), jnp.float32),
        pltpu.VMEM_SHARED((_SPN, _HP), jnp.float32),
        pltpu.VMEM((_CHK,), jnp.int32),
        pltpu.VMEM((_CHK,), jnp.int32),
        pltpu.VMEM((_CHK,), jnp.int32),
        pltpu.VMEM((_CHK,), jnp.int32),
        pltpu.VMEM((_CHK, _HP), jnp.float32),
        pltpu.VMEM((_CHK, _HP), jnp.float32),
        pltpu.SemaphoreType.DMA,
        pltpu.SemaphoreType.DMA,
        pltpu.SemaphoreType.DMA,
        pltpu.SemaphoreType.DMA,
    ],
)
def _sc_conv(h_hbm, src3_hbm, dst3_hbm, zeros_hbm, out_hbm,
             agg, h_spm, sidx0, sidx1, didx0, didx1, rows0, rows1,
             semg0, semg1, semi0, semi1):
    c = lax.axis_index("c")
    s = lax.axis_index("s")
    wid = c * _NSUB + s
    nch = jnp.where(c == _FAST_CORE, _FCH, _SCH)

    pltpu.sync_copy(src3_hbm.at[wid, 0, 0], sidx0)
    pltpu.sync_copy(src3_hbm.at[wid, 1, 0], sidx1)
    pltpu.sync_copy(dst3_hbm.at[wid, 0, 0], didx0)
    pltpu.sync_copy(dst3_hbm.at[wid, 1, 0], didx1)
    pltpu.sync_copy(h_hbm.at[pl.ds(s * _RPS, _RPS)],
                    h_spm.at[pl.ds(s * _RPS, _RPS)])
    pltpu.sync_copy(zeros_hbm.at[pl.ds(s * _RPS, _RPS)],
                    agg.at[pl.ds(s * _RPS, _RPS)])
    plsc.subcore_barrier()
    pltpu.async_copy(h_spm.at[sidx0], rows0, semg0)
    pltpu.async_copy(h_spm.at[sidx1], rows1, semg1)

    def body(t, _):
        g0 = t * 2
        g1 = g0 + 1
        pltpu.make_async_copy(h_spm.at[sidx0], rows0, semg0).wait()
        pltpu.sync_copy(rows0, agg.at[didx0], add=True)

        @pl.when(g0 + 2 < nch)
        def _():
            pltpu.async_copy(src3_hbm.at[wid, g0 + 2, 0], sidx0, semi0)
            pltpu.async_copy(dst3_hbm.at[wid, g0 + 2, 0], didx0, semi0)

        pltpu.make_async_copy(h_spm.at[sidx1], rows1, semg1).wait()
        pltpu.sync_copy(rows1, agg.at[didx1], add=True)

        @pl.when(g1 + 2 < nch)
        def _():
            pltpu.async_copy(src3_hbm.at[wid, g1 + 2, 0], sidx1, semi1)
            pltpu.async_copy(dst3_hbm.at[wid, g1 + 2, 0], didx1, semi1)

        @pl.when(g0 + 2 < nch)
        def _():
            pltpu.make_async_copy(src3_hbm.at[wid, g0 + 2, 0], sidx0,
                                  semi0).wait()
            pltpu.make_async_copy(dst3_hbm.at[wid, g0 + 2, 0], didx0,
                                  semi0).wait()
            pltpu.async_copy(h_spm.at[sidx0], rows0, semg0)

        @pl.when(g1 + 2 < nch)
        def _():
            pltpu.make_async_copy(src3_hbm.at[wid, g1 + 2, 0], sidx1,
                                  semi1).wait()
            pltpu.make_async_copy(dst3_hbm.at[wid, g1 + 2, 0], didx1,
                                  semi1).wait()
            pltpu.async_copy(h_spm.at[sidx1], rows1, semg1)
        return 0
    lax.fori_loop(0, nch // 2, body, 0)
    plsc.subcore_barrier()

    pltpu.sync_copy(agg.at[pl.ds(s * _RPS, _RPS)],
                    out_hbm.at[c, pl.ds(s * _RPS, _RPS)])

    @pl.when(s == _NSUB - 1)
    def _():
        pltpu.sync_copy(zeros_hbm.at[pl.ds(_SPN, _NP - _SPN)],
                        out_hbm.at[c, pl.ds(_SPN, _NP - _SPN)])


def _topk_z(X, scorer):
    R = X.shape[1]
    nrm = jnp.maximum(jnp.sqrt(jnp.sum(scorer * scorer)), 1e-6)
    sw = _mm(X, jnp.broadcast_to(scorer, (R, 128)))
    eye = (lax.broadcasted_iota(jnp.int32, (128, 128), 0)
           == lax.broadcasted_iota(jnp.int32, (128, 128), 1)
           ).astype(jnp.float32)
    st = jnp.concatenate(
        [jnp.sum(sw[a * 128:(a + 1) * 128, :] * eye, axis=0, keepdims=True)
         for a in range(_NP // 128)], axis=0) / nrm
    ni = lax.broadcasted_iota(jnp.int32, (_NP, 1), 0)
    nid = (lax.broadcasted_iota(jnp.int32, (_NP // 128, 128), 0) * 128
           + lax.broadcasted_iota(jnp.int32, (_NP // 128, 128), 1))
    st = jnp.where(nid < _N, st, -jnp.inf)
    kl = lax.broadcasted_iota(jnp.int32, (1, 128), 1)

    def it(k, carry):
        stc, ia, va = carry
        m = jnp.max(stc)
        i = jnp.min(jnp.where(stc == m, nid, _NP))
        ia = jnp.where(kl == k, i, ia)
        va = jnp.where(kl == k, m, va)
        stc = jnp.where(nid == i, -jnp.inf, stc)
        return stc, ia, va

    _, ia, va = lax.fori_loop(
        0, _H, it,
        (st, jnp.zeros((1, 128), jnp.int32), jnp.zeros((1, 128), jnp.float32)))
    sel = ia[:, :_H]
    vals = va[:, :_H]
    Pt = jnp.where(ni == sel, jnp.tanh(vals), 0.0)
    return _mmT(X, Pt)


def _gru(z, w, uw, uu, ub, rw, ru, rb, hw, hu, hb):
    upd = jax.nn.sigmoid(_mm(uw, z) + _mm(uu, w) + ub)
    rst = jax.nn.sigmoid(_mm(rw, z) + _mm(ru, w) + rb)
    hc = jnp.tanh(_mm(hw, z) + _mm(hu, rst * w) + hb)
    return (1.0 - upd) * w + upd * hc


def _pad_w(w):
    return jnp.concatenate([w, jnp.zeros((w.shape[0], _HP - _H), jnp.float32)],
                           axis=1)


def _deg_body(po_ref, pi_ref, qo_ref, qi_ref, dout_ref, din_ref):
    dout_ref[...] = lax.rsqrt(jnp.maximum(po_ref[...] + qo_ref[...], 1.0))
    din_ref[...] = lax.rsqrt(jnp.maximum(pi_ref[...] + qi_ref[...], 1.0))


def _prep_body(x_ref, sc_ref, uw_ref, uu_ref, ub_ref, rw_ref, ru_ref, rb_ref,
               hw_ref, hu_ref, hb_ref, w0_ref, dout_ref,
               w01_ref, h0_ref):
    X = x_ref[...]
    z = _topk_z(X, sc_ref[...])
    w01 = _gru(z, w0_ref[...], uw_ref[...], uu_ref[...], ub_ref[...],
               rw_ref[...], ru_ref[...], rb_ref[...],
               hw_ref[...], hu_ref[...], hb_ref[...])
    w01_ref[...] = w01
    h0_ref[...] = _mm(X * dout_ref[...][:, 0:1], _pad_w(w01))


def _stepb_body(x_ref, dout_ref, sc_ref, uw_ref, uu_ref, ub_ref, rw_ref,
                ru_ref, rb_ref, hw_ref, hu_ref, hb_ref, w01_ref, h1_ref):
    X = x_ref[...]
    z = _topk_z(X, sc_ref[...])
    w02 = _gru(z, w01_ref[...], uw_ref[...], uu_ref[...], ub_ref[...],
               rw_ref[...], ru_ref[...], rb_ref[...],
               hw_ref[...], hu_ref[...], hb_ref[...])
    h1_ref[...] = _mm(X * dout_ref[...][:, 0:1], _pad_w(w02))


def _stepc_body(a0_ref, a1_ref, din_ref, sc_ref,
                uw_ref, uu_ref, ub_ref, rw_ref, ru_ref, rb_ref, hw_ref,
                hu_ref, hb_ref, w1_ref, w11_ref):
    A0 = _rrelu((a0_ref[...] + a1_ref[...]) * din_ref[...][:, 0:1])
    z0 = _topk_z(A0[:, :_H], sc_ref[...])
    w11_ref[...] = _gru(z0, w1_ref[...], uw_ref[...], uu_ref[...], ub_ref[...],
                        rw_ref[...], ru_ref[...], rb_ref[...],
                        hw_ref[...], hu_ref[...], hb_ref[...])


def _stepd_body(b0_ref, b1_ref, din_ref, dout_ref, sc_ref,
                uw_ref, uu_ref, ub_ref, rw_ref, ru_ref, rb_ref, hw_ref,
                hu_ref, hb_ref, w11_ref, h3_ref):
    A1 = _rrelu((b0_ref[...] + b1_ref[...]) * din_ref[...][:, 0:1])
    z1 = _topk_z(A1[:, :_H], sc_ref[...])
    w12 = _gru(z1, w11_ref[...], uw_ref[...], uu_ref[...], ub_ref[...],
               rw_ref[...], ru_ref[...], rb_ref[...],
               hw_ref[...], hu_ref[...], hb_ref[...])
    h3_ref[...] = _mm(A1[:, :_H] * dout_ref[...][:, 0:1], _pad_w(w12))


def _final_body(c0_ref, c1_ref, din_ref, w1_ref, b1_ref, w2_ref, b2_ref,
                out_ref):
    B = _rrelu((c0_ref[...] + c1_ref[...]) * din_ref[...][:, 0:1])[:, :_H]
    hh = jnp.maximum(_mm(B, w1_ref[...]) + b1_ref[...], 0.0)
    w2p = jnp.concatenate(
        [w2_ref[...], jnp.zeros((_CH, 126), jnp.float32)], axis=1)
    out_ref[...] = _mm(hh, w2p)[:, :2] + b2_ref[...]


_sds = jax.ShapeDtypeStruct
_tc_params = pltpu.CompilerParams(vmem_limit_bytes=128 * 1024 * 1024, fuse_transposed_lhs_in_matmul=True)

_tc_deg = pl.pallas_call(
    _deg_body,
    out_shape=(_sds((_NP // 8, 128), jnp.float32),
               _sds((_NP // 8, 128), jnp.float32)),
    compiler_params=_tc_params)

_tc_prep = pl.pallas_call(
    _prep_body,
    out_shape=(_sds((_IN, _H), jnp.float32), _sds((_NP, _HP), jnp.float32)),
    compiler_params=_tc_params)

_tc_stepb = pl.pallas_call(
    _stepb_body, out_shape=_sds((_NP, _HP), jnp.float32),
    compiler_params=_tc_params)

_tc_stepc = pl.pallas_call(
    _stepc_body, out_shape=_sds((_H, _H), jnp.float32),
    compiler_params=_tc_params)

_tc_stepd = pl.pallas_call(
    _stepd_body, out_shape=_sds((_NP, _HP), jnp.float32),
    compiler_params=_tc_params)

_tc_final = pl.pallas_call(
    _final_body, out_shape=_sds((_NP, 2), jnp.float32),
    compiler_params=_tc_params)


def kernel(feat, edge_index, scorer0, gru0_uw, gru0_uu, gru0_ub, gru0_rw,
           gru0_ru, gru0_rb, gru0_hw, gru0_hu, gru0_hb, W0, scorer1, gru1_uw,
           gru1_uu, gru1_ub, gru1_rw, gru1_ru, gru1_rb, gru1_hw, gru1_hu,
           gru1_hb, W1, mlp_w1, mlp_b1, mlp_w2, mlp_b2):
    def _edge_layout(flat):
        nfast = _NSUB * _FCH * _CHK
        fast = flat[:nfast].reshape(_NSUB, _FCH, 1, _CHK)
        slow = jnp.pad(flat[nfast:].reshape(_NSUB, _SCH, 1, _CHK),
                       ((0, 0), (0, _FCH - _SCH), (0, 0), (0, 0)))
        parts = [fast, slow] if _FAST_CORE == 0 else [slow, fast]
        return jnp.concatenate(parts, axis=0)

    pad_e = jnp.full((_EP - _E,), _PAD, jnp.int32)
    srcp = _edge_layout(jnp.concatenate([edge_index[0], pad_e]))
    dstp = _edge_layout(jnp.concatenate([edge_index[1], pad_e]))
    feat0p = jnp.pad(feat[0], ((0, _NP - _N), (0, 0)))
    feat1p = jnp.pad(feat[1], ((0, _NP - _N), (0, 0)))
    zeros80 = jnp.zeros((_NP, _HP), jnp.float32)
    zeros16 = jnp.zeros((_NP, 16), jnp.float32)

    degp = _sc_degrees(srcp, dstp, zeros16)
    dout_pk, din_pk = _tc_deg(
        degp[0, 0].reshape(_NP // 8, 128), degp[0, 1].reshape(_NP // 8, 128),
        degp[1, 0].reshape(_NP // 8, 128), degp[1, 1].reshape(_NP // 8, 128))
    dout16 = dout_pk.reshape(_NP, 16)
    din16 = din_pk.reshape(_NP, 16)
    w01, h0 = _tc_prep(
        feat0p, scorer0, gru0_uw, gru0_uu, gru0_ub, gru0_rw, gru0_ru,
        gru0_rb, gru0_hw, gru0_hu, gru0_hb, W0, dout16)
    aggA = _sc_conv(h0, srcp, dstp, zeros80)
    h1 = _tc_stepb(feat1p, dout16, scorer0, gru0_uw, gru0_uu, gru0_ub,
                   gru0_rw, gru0_ru, gru0_rb, gru0_hw, gru0_hu, gru0_hb, w01)
    aggB = _sc_conv(h1, srcp, dstp, zeros80)
    w11 = _tc_stepc(aggA[0], aggA[1], din16, scorer1, gru1_uw, gru1_uu,
                    gru1_ub, gru1_rw, gru1_ru, gru1_rb, gru1_hw, gru1_hu,
                    gru1_hb, W1)
    h3 = _tc_stepd(aggB[0], aggB[1], din16, dout16, scorer1, gru1_uw,
                   gru1_uu, gru1_ub, gru1_rw, gru1_ru, gru1_rb, gru1_hw,
                   gru1_hu, gru1_hb, w11)
    aggC = _sc_conv(h3, srcp, dstp, zeros80)
    out = _tc_final(aggC[0], aggC[1], din16, mlp_w1,
                    mlp_b1.reshape(1, _CH), mlp_w2, mlp_b2.reshape(1, 2))
    return out[:_N]
